# baseline (speedup 1.0000x reference)
import functools

import jax
import jax.numpy as jnp
from jax import lax
from jax.experimental import pallas as pl
from jax.experimental.pallas import tpu as pltpu
from jax.experimental.pallas import tpu_sc as plsc

N = 10000
E = 320000
D = 128
DH = D // 2

NC = 2
NS = 16
NW = NC * NS

CHUNK = 128
NBUF = 4
NCHUNK = ((E // NS + NBUF * CHUNK - 1) // (NBUF * CHUNK)) * NBUF
EPW = NCHUNK * CHUNK
EPAD = EPW * NS
HC = NCHUNK // NC

NPAD = 10112
ROWS_PER_TILE = NPAD // NS

_mesh = plsc.VectorSubcoreMesh(core_axis_name="c", subcore_axis_name="s")


def _deg_body(dst_hbm, ones_hbm, out_hbm, dst_v, ones_v, acc_sh):
    c = lax.axis_index("c")
    s = lax.axis_index("s")
    pltpu.sync_copy(dst_hbm.at[s], dst_v)
    pltpu.sync_copy(ones_hbm.at[pl.ds(0, CHUNK)], ones_v)
    pltpu.sync_copy(ones_hbm.at[pl.ds(CHUNK, ROWS_PER_TILE)],
                    acc_sh.at[pl.ds(s * ROWS_PER_TILE, ROWS_PER_TILE)])
    plsc.subcore_barrier()

    def body(j, carry):
        pltpu.sync_copy(ones_v, acc_sh.at[dst_v.at[j]], add=True)
        return carry

    lax.fori_loop(c * HC, (c + 1) * HC, body, 0)

    plsc.subcore_barrier()
    pltpu.sync_copy(acc_sh.at[pl.ds(s * ROWS_PER_TILE, ROWS_PER_TILE)],
                    out_hbm.at[c, pl.ds(s * ROWS_PER_TILE, ROWS_PER_TILE)])


def _sc_degree(dst_w, ones_pad):
    return pl.kernel(
        _deg_body,
        out_type=jax.ShapeDtypeStruct((NC, NPAD, 16), jnp.float32),
        mesh=_mesh,
        scratch_types=[
            pltpu.VMEM((NCHUNK, CHUNK), jnp.int32),
            pltpu.VMEM((CHUNK, 16), jnp.float32),
            pltpu.VMEM_SHARED((NPAD, 16), jnp.float32),
        ],
    )(dst_w, ones_pad)


def _agg_body(t2_hbm, src2_hbm, dst_hbm, zeros_hbm, out_hbm,
              src_v, dst_v, r0, r1, r2, r3, acc_sh, g0, g1, g2, g3):
    rows = (r0, r1, r2, r3)
    gsems = (g0, g1, g2, g3)
    c = lax.axis_index("c")
    s = lax.axis_index("s")
    wid = c * NS + s
    pltpu.sync_copy(src2_hbm.at[wid], src_v)
    pltpu.sync_copy(dst_hbm.at[s], dst_v)
    pltpu.sync_copy(zeros_hbm,
                    acc_sh.at[pl.ds(s * ROWS_PER_TILE, ROWS_PER_TILE)])
    plsc.subcore_barrier()

    def gather(j, b):
        pltpu.async_copy(t2_hbm.at[src_v.at[j]], rows[b], gsems[b])

    def gather_wait(j, b):
        pltpu.make_async_copy(t2_hbm.at[src_v.at[j]], rows[b],
                              gsems[b]).wait()

    def scat(j, b):
        pltpu.sync_copy(rows[b], acc_sh.at[dst_v.at[j]], add=True)

    def group(i, carry):
        g = i * NBUF
        for b in range(NBUF):
            gather(g + b, b)
        for b in range(NBUF):
            gather_wait(g + b, b)
            scat(g + b, b)
        return carry

    lax.fori_loop(0, NCHUNK // NBUF, group, 0)

    plsc.subcore_barrier()
    pltpu.sync_copy(acc_sh.at[pl.ds(s * ROWS_PER_TILE, ROWS_PER_TILE)],
                    out_hbm.at[c, pl.ds(s * ROWS_PER_TILE, ROWS_PER_TILE)])


def _sc_aggregate(t2, src2_w, dst_w, zeros_rows):
    return pl.kernel(
        _agg_body,
        out_type=jax.ShapeDtypeStruct((NC, NPAD, DH), jnp.float32),
        mesh=_mesh,
        compiler_params=pltpu.CompilerParams(use_tc_tiling_on_sc=False),
        scratch_types=[
            pltpu.VMEM((NCHUNK, CHUNK), jnp.int32),
            pltpu.VMEM((NCHUNK, CHUNK), jnp.int32),
            pltpu.VMEM((CHUNK, DH), jnp.float32),
            pltpu.VMEM((CHUNK, DH), jnp.float32),
            pltpu.VMEM((CHUNK, DH), jnp.float32),
            pltpu.VMEM((CHUNK, DH), jnp.float32),
            pltpu.VMEM_SHARED((NPAD, DH), jnp.float32),
            pltpu.SemaphoreType.DMA,
            pltpu.SemaphoreType.DMA,
            pltpu.SemaphoreType.DMA,
            pltpu.SemaphoreType.DMA,
        ],
    )(t2, src2_w, dst_w, zeros_rows)


_RB = 1000
_GRID = N // _RB


def _dinv_body(p_ref, o_ref):
    deg = p_ref[0, :, 0:1] + p_ref[1, :, 0:1] + 1.0
    o_ref[...] = jnp.broadcast_to(lax.rsqrt(deg), (_RB, D))


def _tc_dinv(deg_p):
    return pl.pallas_call(
        _dinv_body,
        grid=(_GRID,),
        in_specs=[pl.BlockSpec((NC, _RB, 16), lambda i: (0, i, 0))],
        out_specs=pl.BlockSpec((_RB, D), lambda i: (i, 0)),
        out_shape=jax.ShapeDtypeStruct((N, D), jnp.float32),
    )(deg_p)


def _matmul_body(x_ref, w_ref, dinv_ref, o_ref):
    h = jnp.dot(x_ref[...], w_ref[...], preferred_element_type=jnp.float32)
    o_ref[...] = dinv_ref[...] * h


def _tc_matmul(x, w, dinv_b):
    return pl.pallas_call(
        _matmul_body,
        grid=(_GRID,),
        in_specs=[
            pl.BlockSpec((_RB, D), lambda i: (i, 0)),
            pl.BlockSpec((D, D), lambda i: (0, 0)),
            pl.BlockSpec((_RB, D), lambda i: (i, 0)),
        ],
        out_specs=pl.BlockSpec((_RB, D), lambda i: (i, 0)),
        out_shape=jax.ShapeDtypeStruct((N, D), jnp.float32),
    )(x, w, dinv_b)


def _combine_body(relu, p0_ref, p1_ref, t_ref, dinv_ref, b_ref, o_ref):
    agg = jnp.concatenate([p0_ref[0], p1_ref[0]], axis=1)
    out = dinv_ref[...] * (agg + t_ref[...]) + b_ref[...]
    if relu:
        out = jnp.maximum(out, 0.0)
    o_ref[...] = out


def _tc_combine(p, t, dinv_b, b, relu):
    return pl.pallas_call(
        functools.partial(_combine_body, relu),
        grid=(_GRID,),
        in_specs=[
            pl.BlockSpec((1, _RB, DH), lambda i: (0, i, 0)),
            pl.BlockSpec((1, _RB, DH), lambda i: (1, i, 0)),
            pl.BlockSpec((_RB, D), lambda i: (i, 0)),
            pl.BlockSpec((_RB, D), lambda i: (i, 0)),
            pl.BlockSpec((1, D), lambda i: (0, 0)),
        ],
        out_specs=pl.BlockSpec((_RB, D), lambda i: (i, 0)),
        out_shape=jax.ShapeDtypeStruct((N, D), jnp.float32),
    )(p, p, t, dinv_b, b)


def kernel(x, edge_index, W1, b1, W2, b2, W3, b3):
    src = edge_index[0]
    dst = edge_index[1]
    pad = EPAD - E
    srcp = jnp.concatenate(
        [src, jnp.zeros((pad,), jnp.int32)]).reshape(1, NS, NCHUNK, CHUNK)
    half = jnp.arange(NC, dtype=jnp.int32).reshape(NC, 1, 1, 1)
    src2_w = (2 * srcp + half).reshape(NW, NCHUNK, CHUNK)
    dst_w = jnp.concatenate(
        [dst, jnp.full((pad,), N, jnp.int32)]).reshape(NS, NCHUNK, CHUNK)

    ones_pad = jnp.concatenate([
        jnp.ones((CHUNK, 16), jnp.float32),
        jnp.zeros((ROWS_PER_TILE, 16), jnp.float32)])
    zeros_rows = jnp.zeros((ROWS_PER_TILE, DH), jnp.float32)

    deg_p = _sc_degree(dst_w, ones_pad)
    dinv_b = _tc_dinv(deg_p)

    h = x
    for (W, b, relu) in ((W1, b1, True), (W2, b2, True), (W3, b3, False)):
        t = _tc_matmul(h, W, dinv_b)
        p = _sc_aggregate(t.reshape(2 * N, DH), src2_w, dst_w, zeros_rows)
        h = _tc_combine(p, t, dinv_b, b.reshape(1, D), relu)
    return h
